# baseline (speedup 1.0000x reference)
import functools

import jax
import jax.numpy as jnp
from jax import lax
from jax.experimental import pallas as pl
from jax.experimental.pallas import tpu as pltpu
from jax.experimental.pallas import tpu_sc as plsc

NC = 2
NS = 16
L = 16

MAXD = 32
D = 16
PROWS = 34
SIGN_OFF = 10 * PROWS
TROWS = SIGN_OFF + 3


def _build_sc_call(n_rows):
    nw = NC * NS
    rpw = n_rows // nw

    mesh = plsc.VectorSubcoreMesh(
        core_axis_name="c", subcore_axis_name="s",
        num_cores=NC, num_subcores=NS)

    @functools.partial(
        pl.kernel,
        out_type=jax.ShapeDtypeStruct((n_rows * D,), jnp.float32),
        mesh=mesh,
        compiler_params=pltpu.CompilerParams(needs_layout_passes=False),
        scratch_types=[
            pltpu.VMEM((10 * D,), jnp.float32),
            pltpu.VMEM((PROWS * D,), jnp.float32),
            pltpu.VMEM((3 * D,), jnp.float32),
            pltpu.VMEM((TROWS * D,), jnp.float32),
            pltpu.VMEM((rpw * MAXD,), jnp.int32),
            pltpu.VMEM((rpw * MAXD,), jnp.int32),
            pltpu.VMEM((rpw,), jnp.int32),
            pltpu.VMEM((rpw * D,), jnp.float32),
            pltpu.SemaphoreType.DMA,
        ],
    )
    def sc_kernel(dt_hbm, pt_hbm, st_hbm, dig_hbm, pos_hbm, sgn_hbm, out_hbm,
                  dt_v, pt_v, st_v, tbl_v, dig_v, pos_v, sgn_v, out_v, sem):
        wid = lax.axis_index("s") * NC + lax.axis_index("c")
        r0 = wid * rpw

        cps = [
            pltpu.async_copy(dt_hbm, dt_v, sem),
            pltpu.async_copy(pt_hbm, pt_v, sem),
            pltpu.async_copy(st_hbm, st_v, sem),
            pltpu.async_copy(
                dig_hbm.at[pl.ds(r0 * MAXD, rpw * MAXD)], dig_v, sem),
            pltpu.async_copy(
                pos_hbm.at[pl.ds(r0 * MAXD, rpw * MAXD)], pos_v, sem),
            pltpu.async_copy(sgn_hbm.at[pl.ds(r0, rpw)], sgn_v, sem),
        ]
        for cp in cps:
            cp.wait()

        for d in range(10):
            drow = dt_v[pl.ds(d * D, D)]
            for p in range(PROWS):
                tbl_v[pl.ds((d * PROWS + p) * D, D)] = (
                    drow + pt_v[pl.ds(p * D, D)])
        for s in range(3):
            tbl_v[pl.ds((SIGN_OFF + s) * D, D)] = st_v[pl.ds(s * D, D)]

        lanes = lax.iota(jnp.int32, L)

        @plsc.parallel_loop(0, rpw, 1, unroll=2)
        def row_body(r):
            d0 = dig_v[pl.ds(r * MAXD, L)]
            d1 = dig_v[pl.ds(r * MAXD + L, L)]
            p0 = pos_v[pl.ds(r * MAXD, L)]
            p1 = pos_v[pl.ds(r * MAXD + L, L)]
            i0 = (d0 * PROWS + p0) * D
            i1 = (d1 * PROWS + p1) * D

            r_splat = jnp.broadcast_to(r, (L,)).astype(jnp.int32)
            s_splat = plsc.load_gather(sgn_v, [r_splat])
            accs = [plsc.load_gather(tbl_v, [s_splat * D + SIGN_OFF * D + lanes]),
                    None, None, None]

            k = 0
            for half in (i0, i1):
                for i in range(L):
                    base = jnp.take_along_axis(
                        half, jnp.full((L,), i, jnp.int32), axis=0)
                    g = plsc.load_gather(tbl_v, [base + lanes])
                    a = k % 4
                    accs[a] = g if accs[a] is None else accs[a] + g
                    k += 1

            out_v[pl.ds(r * D, D)] = (accs[0] + accs[1]) + (accs[2] + accs[3])
        pltpu.sync_copy(out_v, out_hbm.at[pl.ds(r0 * D, rpw * D)])

    return sc_kernel


def kernel(x, digit_table, sign_table, pos_table, digits, positions, signs):
    b, c, s, _ = x.shape
    n = b * c * s
    nw = NC * NS
    if n == 16384:
        splits = [4096, 12288]
    elif n % nw == 0:
        splits = [n]
    else:
        raise ValueError(f"batch {n} not divisible by {nw}")
    dt = digit_table.reshape(10 * D)
    pt = pos_table.reshape(PROWS * D)
    st = sign_table.reshape(3 * D)
    outs = []
    a = 0
    for m in splits:
        call = _build_sc_call(m)
        outs.append(
            call(dt, pt, st,
                 digits[a:a + m].reshape(m * MAXD),
                 positions[a:a + m].reshape(m * MAXD),
                 signs[a:a + m]))
        a += m
    out = outs[0] if len(outs) == 1 else jnp.concatenate(outs)
    return out.reshape(b, c, s, D)


# speedup vs baseline: 1.2305x; 1.2305x over previous
import functools

import jax
import jax.numpy as jnp
from jax import lax
from jax.experimental import pallas as pl
from jax.experimental.pallas import tpu as pltpu
from jax.experimental.pallas import tpu_sc as plsc

NC = 2
NS = 16
L = 16

MAXD = 32
D = 16
PROWS = 34
SIGN_OFF = 10 * PROWS
TROWS = SIGN_OFF + 3


def _build_sc_call(n_rows):
    nw = NC * NS
    rpw = n_rows // nw

    mesh = plsc.VectorSubcoreMesh(
        core_axis_name="c", subcore_axis_name="s",
        num_cores=NC, num_subcores=NS)

    @functools.partial(
        pl.kernel,
        out_type=jax.ShapeDtypeStruct((n_rows * D,), jnp.float32),
        mesh=mesh,
        compiler_params=pltpu.CompilerParams(needs_layout_passes=False),
        scratch_types=[
            pltpu.VMEM((10 * D,), jnp.float32),
            pltpu.VMEM((PROWS * D,), jnp.float32),
            pltpu.VMEM((3 * D,), jnp.float32),
            pltpu.VMEM((TROWS * D,), jnp.float32),
            pltpu.VMEM((rpw * MAXD,), jnp.int32),
            pltpu.VMEM((rpw * MAXD,), jnp.int32),
            pltpu.VMEM((rpw,), jnp.int32),
            pltpu.VMEM((rpw * D,), jnp.float32),
            pltpu.SemaphoreType.DMA,
        ],
    )
    def sc_kernel(dt_hbm, pt_hbm, st_hbm, dig_hbm, pos_hbm, sgn_hbm, out_hbm,
                  dt_v, pt_v, st_v, tbl_v, dig_v, pos_v, sgn_v, out_v, sem):
        wid = lax.axis_index("s") * NC + lax.axis_index("c")
        r0 = wid * rpw

        cps = [
            pltpu.async_copy(dt_hbm, dt_v, sem),
            pltpu.async_copy(pt_hbm, pt_v, sem),
            pltpu.async_copy(st_hbm, st_v, sem),
            pltpu.async_copy(
                dig_hbm.at[pl.ds(r0 * MAXD, rpw * MAXD)], dig_v, sem),
            pltpu.async_copy(
                pos_hbm.at[pl.ds(r0 * MAXD, rpw * MAXD)], pos_v, sem),
            pltpu.async_copy(sgn_hbm.at[pl.ds(r0, rpw)], sgn_v, sem),
        ]
        for cp in cps[:3]:
            cp.wait()

        for d in range(10):
            drow = dt_v[pl.ds(d * D, D)]
            for p in range(PROWS):
                tbl_v[pl.ds((d * PROWS + p) * D, D)] = (
                    drow + pt_v[pl.ds(p * D, D)])
        for s in range(3):
            tbl_v[pl.ds((SIGN_OFF + s) * D, D)] = st_v[pl.ds(s * D, D)]
        for cp in cps[3:]:
            cp.wait()

        lanes = lax.iota(jnp.int32, L)

        @plsc.parallel_loop(0, rpw, 1, unroll=2)
        def row_body(r):
            d0 = dig_v[pl.ds(r * MAXD, L)]
            d1 = dig_v[pl.ds(r * MAXD + L, L)]
            p0 = pos_v[pl.ds(r * MAXD, L)]
            p1 = pos_v[pl.ds(r * MAXD + L, L)]
            i0 = (d0 * PROWS + p0) * D
            i1 = (d1 * PROWS + p1) * D

            r_splat = jnp.broadcast_to(r, (L,)).astype(jnp.int32)
            s_splat = plsc.load_gather(sgn_v, [r_splat])
            accs = [plsc.load_gather(tbl_v, [s_splat * D + SIGN_OFF * D + lanes]),
                    None, None, None]

            k = 0
            for half in (i0, i1):
                for i in range(L):
                    base = jnp.take_along_axis(
                        half, jnp.full((L,), i, jnp.int32), axis=0)
                    g = plsc.load_gather(tbl_v, [base + lanes])
                    a = k % 4
                    accs[a] = g if accs[a] is None else accs[a] + g
                    k += 1

            out_v[pl.ds(r * D, D)] = (accs[0] + accs[1]) + (accs[2] + accs[3])
        pltpu.sync_copy(out_v, out_hbm.at[pl.ds(r0 * D, rpw * D)])

    return sc_kernel


def kernel(x, digit_table, sign_table, pos_table, digits, positions, signs):
    b, c, s, _ = x.shape
    n = b * c * s
    nw = NC * NS
    if n % (2 * nw) == 0:
        splits = [n // 2, n // 2]
    elif n % nw == 0:
        splits = [n]
    else:
        raise ValueError(f"batch {n} not divisible by {nw}")
    dt = digit_table.reshape(10 * D)
    pt = pos_table.reshape(PROWS * D)
    st = sign_table.reshape(3 * D)
    calls = {}
    outs = []
    a = 0
    for m in splits:
        if m not in calls:
            calls[m] = _build_sc_call(m)
        call = calls[m]
        outs.append(
            call(dt, pt, st,
                 digits[a:a + m].reshape(m * MAXD),
                 positions[a:a + m].reshape(m * MAXD),
                 signs[a:a + m]))
        a += m
    out = outs[0] if len(outs) == 1 else jnp.concatenate(outs)
    return out.reshape(b, c, s, D)


# speedup vs baseline: 1.2312x; 1.0006x over previous
import functools

import jax
import jax.numpy as jnp
from jax import lax
from jax.experimental import pallas as pl
from jax.experimental.pallas import tpu as pltpu
from jax.experimental.pallas import tpu_sc as plsc

NC = 2
NS = 16
L = 16

MAXD = 32
D = 16
PROWS = 34
SIGN_OFF = 10 * PROWS
TROWS = SIGN_OFF + 3


def _build_sc_call(n_rows):
    nw = NC * NS
    rpw = n_rows // nw

    mesh = plsc.VectorSubcoreMesh(
        core_axis_name="c", subcore_axis_name="s",
        num_cores=NC, num_subcores=NS)

    @functools.partial(
        pl.kernel,
        out_type=jax.ShapeDtypeStruct((n_rows * D,), jnp.float32),
        mesh=mesh,
        compiler_params=pltpu.CompilerParams(needs_layout_passes=False),
        scratch_types=[
            pltpu.VMEM((10 * D,), jnp.float32),
            pltpu.VMEM((PROWS * D,), jnp.float32),
            pltpu.VMEM((3 * D,), jnp.float32),
            pltpu.VMEM((TROWS * D,), jnp.float32),
            pltpu.VMEM((rpw * MAXD,), jnp.int32),
            pltpu.VMEM((rpw * MAXD,), jnp.int32),
            pltpu.VMEM((rpw,), jnp.int32),
            pltpu.VMEM((rpw * D,), jnp.float32),
            pltpu.SemaphoreType.DMA,
        ],
    )
    def sc_kernel(dt_hbm, pt_hbm, st_hbm, dig_hbm, pos_hbm, sgn_hbm, out_hbm,
                  dt_v, pt_v, st_v, tbl_v, dig_v, pos_v, sgn_v, out_v, sem):
        wid = lax.axis_index("s") * NC + lax.axis_index("c")
        r0 = wid * rpw

        cps = [
            pltpu.async_copy(dt_hbm, dt_v, sem),
            pltpu.async_copy(pt_hbm, pt_v, sem),
            pltpu.async_copy(st_hbm, st_v, sem),
            pltpu.async_copy(
                dig_hbm.at[pl.ds(r0 * MAXD, rpw * MAXD)], dig_v, sem),
            pltpu.async_copy(
                pos_hbm.at[pl.ds(r0 * MAXD, rpw * MAXD)], pos_v, sem),
            pltpu.async_copy(sgn_hbm.at[pl.ds(r0, rpw)], sgn_v, sem),
        ]
        for cp in cps[:3]:
            cp.wait()

        for d in range(10):
            drow = dt_v[pl.ds(d * D, D)]
            for p in range(PROWS):
                tbl_v[pl.ds((d * PROWS + p) * D, D)] = (
                    drow + pt_v[pl.ds(p * D, D)])
        for s in range(3):
            tbl_v[pl.ds((SIGN_OFF + s) * D, D)] = st_v[pl.ds(s * D, D)]
        for cp in cps[3:]:
            cp.wait()

        lanes = lax.iota(jnp.int32, L)

        @plsc.parallel_loop(0, rpw, 1, unroll=4)
        def row_body(r):
            d0 = dig_v[pl.ds(r * MAXD, L)]
            d1 = dig_v[pl.ds(r * MAXD + L, L)]
            p0 = pos_v[pl.ds(r * MAXD, L)]
            p1 = pos_v[pl.ds(r * MAXD + L, L)]
            i0 = (d0 * PROWS + p0) * D
            i1 = (d1 * PROWS + p1) * D

            r_splat = jnp.broadcast_to(r, (L,)).astype(jnp.int32)
            s_splat = plsc.load_gather(sgn_v, [r_splat])
            accs = [plsc.load_gather(tbl_v, [s_splat * D + SIGN_OFF * D + lanes]),
                    None, None, None]

            k = 0
            for half in (i0, i1):
                for i in range(L):
                    base = jnp.take_along_axis(
                        half, jnp.full((L,), i, jnp.int32), axis=0)
                    g = plsc.load_gather(tbl_v, [base + lanes])
                    a = k % 4
                    accs[a] = g if accs[a] is None else accs[a] + g
                    k += 1

            out_v[pl.ds(r * D, D)] = (accs[0] + accs[1]) + (accs[2] + accs[3])
        pltpu.sync_copy(out_v, out_hbm.at[pl.ds(r0 * D, rpw * D)])

    return sc_kernel


def kernel(x, digit_table, sign_table, pos_table, digits, positions, signs):
    b, c, s, _ = x.shape
    n = b * c * s
    nw = NC * NS
    if n % (2 * nw) == 0:
        splits = [n // 2, n // 2]
    elif n % nw == 0:
        splits = [n]
    else:
        raise ValueError(f"batch {n} not divisible by {nw}")
    dt = digit_table.reshape(10 * D)
    pt = pos_table.reshape(PROWS * D)
    st = sign_table.reshape(3 * D)
    calls = {}
    outs = []
    a = 0
    for m in splits:
        if m not in calls:
            calls[m] = _build_sc_call(m)
        call = calls[m]
        outs.append(
            call(dt, pt, st,
                 digits[a:a + m].reshape(m * MAXD),
                 positions[a:a + m].reshape(m * MAXD),
                 signs[a:a + m]))
        a += m
    out = outs[0] if len(outs) == 1 else jnp.concatenate(outs)
    return out.reshape(b, c, s, D)
